# baseline (speedup 1.0000x reference)
import functools
import jax
import jax.numpy as jnp
from jax import lax
from jax.experimental import pallas as pl
from jax.experimental.pallas import tpu as pltpu
from jax.experimental.pallas import tpu_sc as plsc


def _topk_weights_kernel(lp_ref, hpt_ref, idx_ref, w_ref, *, row_offset):
    lp = lp_ref[...]
    hpt = hpt_ref[...]
    bq = lp.shape[0]
    m = hpt.shape[1]

    lpb = lp.astype(jnp.bfloat16).astype(jnp.float32)
    hptb = hpt.astype(jnp.bfloat16).astype(jnp.float32)
    s = lpb[:, 0:1] * hptb[0:1, :]
    s = s + lpb[:, 1:2] * hptb[1:2, :]
    s = s + lpb[:, 2:3] * hptb[2:3, :]
    d = -2.0 * s
    lpsq = lp[:, 0:1] * lp[:, 0:1]
    lpsq = lpsq + lp[:, 1:2] * lp[:, 1:2]
    lpsq = lpsq + lp[:, 2:3] * lp[:, 2:3]
    hpsq = hpt[0:1, :] * hpt[0:1, :]
    hpsq = hpsq + hpt[1:2, :] * hpt[1:2, :]
    hpsq = hpsq + hpt[2:3, :] * hpt[2:3, :]
    d = d + lpsq
    d = d + hpsq

    iota = lax.broadcasted_iota(jnp.int32, (bq, m), 1)
    inf = jnp.float32(jnp.inf)
    cur = d
    vals, idxs = [], []
    for _ in range(3):
        mv = jnp.min(cur, axis=1, keepdims=True)
        mi = jnp.min(jnp.where(cur == mv, iota, m), axis=1, keepdims=True)
        vals.append(mv)
        idxs.append(mi)
        cur = jnp.where(iota == mi, inf, cur)

    recip = [1.0 / (v + 1e-8) for v in vals]
    norm = recip[0] + recip[1] + recip[2]
    w = [r / norm for r in recip]

    idx_ref[...] = jnp.concatenate(idxs, axis=1) + row_offset
    w_ref[...] = jnp.concatenate(
        [jnp.broadcast_to(wk, (bq, 16)) for wk in w], axis=1)


def _tc_stage_batch(lp_b, hpt_b, M, row_offset):
    N, _ = lp_b.shape
    BQ = 1024
    return pl.pallas_call(
        functools.partial(_topk_weights_kernel, row_offset=row_offset),
        grid=(N // BQ,),
        in_specs=[
            pl.BlockSpec((BQ, 3), lambda i: (i, 0)),
            pl.BlockSpec((3, M), lambda i: (0, 0)),
        ],
        out_specs=[
            pl.BlockSpec((BQ, 3), lambda i: (i, 0)),
            pl.BlockSpec((BQ, 48), lambda i: (i, 0)),
        ],
        out_shape=[
            jax.ShapeDtypeStruct((N, 3), jnp.int32),
            jax.ShapeDtypeStruct((N, 48), jnp.float32),
        ],
    )(lp_b, hpt_b)


_NC, _NS, _LANES = 2, 16, 16
_NW = _NC * _NS


def _make_sc_gather(Q, C, CQ):
    qpw = Q // _NW
    nchunk = qpw // CQ
    assert nchunk % 2 == 0
    mesh = plsc.VectorSubcoreMesh(core_axis_name="c", subcore_axis_name="s")

    @functools.partial(
        pl.kernel,
        out_type=jax.ShapeDtypeStruct((Q, C), jnp.float32),
        mesh=mesh,
        scratch_types=[
            pltpu.VMEM((3 * qpw,), jnp.int32),
            pltpu.VMEM((CQ, 48), jnp.float32),
            pltpu.VMEM((CQ, 48), jnp.float32),
            pltpu.VMEM((3 * CQ, C), jnp.float32),
            pltpu.VMEM((3 * CQ, C), jnp.float32),
            pltpu.VMEM((CQ, C), jnp.float32),
            pltpu.VMEM((CQ, C), jnp.float32),
            pltpu.SemaphoreType.DMA,
            pltpu.SemaphoreType.DMA,
            pltpu.SemaphoreType.DMA,
            pltpu.SemaphoreType.DMA,
            pltpu.SemaphoreType.DMA,
            pltpu.SemaphoreType.DMA,
        ],
    )
    def sc_gather(feats_hbm, idx_hbm, w_hbm, out_hbm,
                  idx_v, w_va, w_vb, rows_a, rows_b, out_va, out_vb,
                  gsem_a, gsem_b, wsem_a, wsem_b, osem_a, osem_b):
        wid = lax.axis_index("s") * _NC + lax.axis_index("c")
        base_q = wid * qpw

        pltpu.sync_copy(idx_hbm.at[pl.ds(base_q * 3, 3 * qpw)], idx_v)

        def chunk_start(cc, rows, w_v, gsem, wsem):
            pltpu.async_copy(
                feats_hbm.at[idx_v.at[pl.ds(cc * 3 * CQ, 3 * CQ)]],
                rows, gsem)
            pltpu.async_copy(
                w_hbm.at[pl.ds(base_q + cc * CQ, CQ)], w_v, wsem)

        def chunk_wait(cc, rows, w_v, gsem, wsem):
            pltpu.make_async_copy(
                feats_hbm.at[idx_v.at[pl.ds(cc * 3 * CQ, 3 * CQ)]],
                rows, gsem).wait()
            pltpu.make_async_copy(
                w_hbm.at[pl.ds(base_q + cc * CQ, CQ)], w_v, wsem).wait()

        def out_wait(cc, out_v, osem):
            pltpu.make_async_copy(
                out_v, out_hbm.at[pl.ds(base_q + cc * CQ, CQ)], osem).wait()

        def compute_store(cc, rows, w_v, out_v, osem):
            @pl.when(cc >= 2)
            def _():
                out_wait(cc - 2, out_v, osem)

            for q in range(CQ):
                w0 = w_v[q, pl.ds(0, 16)]
                w1 = w_v[q, pl.ds(16, 16)]
                w2 = w_v[q, pl.ds(32, 16)]
                for dc in range(C // _LANES):
                    sl = pl.ds(dc * _LANES, _LANES)
                    acc = rows[3 * q, sl] * w0
                    acc = acc + rows[3 * q + 1, sl] * w1
                    acc = acc + rows[3 * q + 2, sl] * w2
                    out_v[q, sl] = acc
            pltpu.async_copy(
                out_v, out_hbm.at[pl.ds(base_q + cc * CQ, CQ)], osem)

        chunk_start(0, rows_a, w_va, gsem_a, wsem_a)
        chunk_start(1, rows_b, w_vb, gsem_b, wsem_b)

        def pair_body(i, carry):
            cc = 2 * i
            chunk_wait(cc, rows_a, w_va, gsem_a, wsem_a)
            compute_store(cc, rows_a, w_va, out_va, osem_a)

            @pl.when(cc + 2 < nchunk)
            def _():
                chunk_start(cc + 2, rows_a, w_va, gsem_a, wsem_a)

            chunk_wait(cc + 1, rows_b, w_vb, gsem_b, wsem_b)
            compute_store(cc + 1, rows_b, w_vb, out_vb, osem_b)

            @pl.when(cc + 3 < nchunk)
            def _():
                chunk_start(cc + 3, rows_b, w_vb, gsem_b, wsem_b)

            return carry

        lax.fori_loop(0, nchunk // 2, pair_body, 0)
        out_wait(nchunk - 2, out_va, osem_a)
        out_wait(nchunk - 1, out_vb, osem_b)

    return sc_gather


def kernel(higher_feats, lower_points, higher_points):
    B, N, _ = lower_points.shape
    _, M, C = higher_feats.shape
    feats_flat = higher_feats.reshape(B * M, C)
    hpt = jnp.swapaxes(higher_points, 1, 2)
    H = 2
    NS = N // H
    sc_fn = _make_sc_gather(NS, C, 16)
    outs = []
    for b in range(B):
        for h in range(H):
            lp_s = lax.slice_in_dim(lower_points[b], h * NS, (h + 1) * NS)
            idx_s, w_s = _tc_stage_batch(lp_s, hpt[b], M, b * M)
            outs.append(sc_fn(feats_flat, idx_s.reshape(NS * 3),
                              w_s.reshape(NS, 48)))
    return jnp.concatenate(outs, axis=0).reshape(B, N, C)


# speedup vs baseline: 1.1104x; 1.1104x over previous
import functools
import jax
import jax.numpy as jnp
from jax import lax
from jax.experimental import pallas as pl
from jax.experimental.pallas import tpu as pltpu
from jax.experimental.pallas import tpu_sc as plsc


def _topk_weights_kernel(lp_ref, hpt_ref, idx_ref, w_ref, *, row_offset):
    lp = lp_ref[...]
    hpt = hpt_ref[...]
    bq = lp.shape[0]
    m = hpt.shape[1]

    lpb = lp.astype(jnp.bfloat16).astype(jnp.float32)
    hptb = hpt.astype(jnp.bfloat16).astype(jnp.float32)
    s = lpb[:, 0:1] * hptb[0:1, :]
    s = s + lpb[:, 1:2] * hptb[1:2, :]
    s = s + lpb[:, 2:3] * hptb[2:3, :]
    d = -2.0 * s
    lpsq = lp[:, 0:1] * lp[:, 0:1]
    lpsq = lpsq + lp[:, 1:2] * lp[:, 1:2]
    lpsq = lpsq + lp[:, 2:3] * lp[:, 2:3]
    hpsq = hpt[0:1, :] * hpt[0:1, :]
    hpsq = hpsq + hpt[1:2, :] * hpt[1:2, :]
    hpsq = hpsq + hpt[2:3, :] * hpt[2:3, :]
    d = d + lpsq
    d = d + hpsq

    iota = lax.broadcasted_iota(jnp.int32, (bq, m), 1).astype(jnp.float32)
    mf = jnp.float32(m)
    inf = jnp.float32(jnp.inf)
    cur = d
    vals, idxs = [], []
    for _ in range(3):
        mv = jnp.min(cur, axis=1, keepdims=True)
        mi = jnp.min(jnp.where(cur == mv, iota, mf), axis=1, keepdims=True)
        vals.append(mv)
        idxs.append(mi)
        cur = jnp.where(iota == mi, inf, cur)

    recip = [1.0 / (v + 1e-8) for v in vals]
    norm = recip[0] + recip[1] + recip[2]
    w = [r / norm for r in recip]

    idx_ref[...] = (jnp.concatenate(idxs, axis=1).astype(jnp.int32)
                    + row_offset)
    w_ref[...] = jnp.concatenate(
        [jnp.broadcast_to(wk, (bq, 16)) for wk in w], axis=1)


def _tc_stage_batch(lp_b, hpt_b, M, row_offset):
    N, _ = lp_b.shape
    BQ = 1024
    return pl.pallas_call(
        functools.partial(_topk_weights_kernel, row_offset=row_offset),
        grid=(N // BQ,),
        in_specs=[
            pl.BlockSpec((BQ, 3), lambda i: (i, 0)),
            pl.BlockSpec((3, M), lambda i: (0, 0)),
        ],
        out_specs=[
            pl.BlockSpec((BQ, 3), lambda i: (i, 0)),
            pl.BlockSpec((BQ, 48), lambda i: (i, 0)),
        ],
        out_shape=[
            jax.ShapeDtypeStruct((N, 3), jnp.int32),
            jax.ShapeDtypeStruct((N, 48), jnp.float32),
        ],
    )(lp_b, hpt_b)


_NC, _NS, _LANES = 2, 16, 16
_NW = _NC * _NS


def _make_sc_gather(Q, C, CQ, NBUF):
    qpw = Q // _NW
    nchunk = qpw // CQ
    assert nchunk % NBUF == 0
    mesh = plsc.VectorSubcoreMesh(core_axis_name="c", subcore_axis_name="s")

    scratch = (
        [pltpu.VMEM((3 * qpw,), jnp.int32)]
        + [pltpu.VMEM((CQ, 48), jnp.float32) for _ in range(NBUF)]
        + [pltpu.VMEM((3 * CQ, C), jnp.float32) for _ in range(NBUF)]
        + [pltpu.VMEM((CQ, C), jnp.float32) for _ in range(NBUF)]
        + [pltpu.SemaphoreType.DMA for _ in range(3 * NBUF)]
    )

    @functools.partial(
        pl.kernel,
        out_type=jax.ShapeDtypeStruct((Q, C), jnp.float32),
        mesh=mesh,
        scratch_types=scratch,
    )
    def sc_gather(feats_hbm, idx_hbm, w_hbm, out_hbm, idx_v, *bufs):
        w_vs = bufs[0:NBUF]
        rows_vs = bufs[NBUF:2 * NBUF]
        out_vs = bufs[2 * NBUF:3 * NBUF]
        gsems = bufs[3 * NBUF:4 * NBUF]
        wsems = bufs[4 * NBUF:5 * NBUF]
        osems = bufs[5 * NBUF:6 * NBUF]

        wid = lax.axis_index("s") * _NC + lax.axis_index("c")
        base_q = wid * qpw

        pltpu.sync_copy(idx_hbm.at[pl.ds(base_q * 3, 3 * qpw)], idx_v)

        def chunk_start(cc, r):
            pltpu.async_copy(
                feats_hbm.at[idx_v.at[pl.ds(cc * 3 * CQ, 3 * CQ)]],
                rows_vs[r], gsems[r])
            pltpu.async_copy(
                w_hbm.at[pl.ds(base_q + cc * CQ, CQ)], w_vs[r], wsems[r])

        def chunk_wait(cc, r):
            pltpu.make_async_copy(
                feats_hbm.at[idx_v.at[pl.ds(cc * 3 * CQ, 3 * CQ)]],
                rows_vs[r], gsems[r]).wait()
            pltpu.make_async_copy(
                w_hbm.at[pl.ds(base_q + cc * CQ, CQ)], w_vs[r],
                wsems[r]).wait()

        def out_wait(cc, r):
            pltpu.make_async_copy(
                out_vs[r], out_hbm.at[pl.ds(base_q + cc * CQ, CQ)],
                osems[r]).wait()

        def compute_store(cc, r):
            @pl.when(cc >= NBUF)
            def _():
                out_wait(cc - NBUF, r)

            rows = rows_vs[r]
            w_v = w_vs[r]
            out_v = out_vs[r]
            for q in range(CQ):
                w0 = w_v[q, pl.ds(0, 16)]
                w1 = w_v[q, pl.ds(16, 16)]
                w2 = w_v[q, pl.ds(32, 16)]
                for dc in range(C // _LANES):
                    sl = pl.ds(dc * _LANES, _LANES)
                    acc = rows[3 * q, sl] * w0
                    acc = acc + rows[3 * q + 1, sl] * w1
                    acc = acc + rows[3 * q + 2, sl] * w2
                    out_v[q, sl] = acc
            pltpu.async_copy(
                out_v, out_hbm.at[pl.ds(base_q + cc * CQ, CQ)], osems[r])

        for r in range(NBUF):
            chunk_start(r, r)

        def ring_body(i, carry):
            for r in range(NBUF):
                cc = NBUF * i + r
                chunk_wait(cc, r)
                compute_store(cc, r)

                @pl.when(cc + NBUF < nchunk)
                def _():
                    chunk_start(cc + NBUF, r)

            return carry

        lax.fori_loop(0, nchunk // NBUF, ring_body, 0)
        for r in range(NBUF):
            out_wait(nchunk - NBUF + r, r)

    return sc_gather


def kernel(higher_feats, lower_points, higher_points):
    B, N, _ = lower_points.shape
    _, M, C = higher_feats.shape
    feats_flat = higher_feats.reshape(B * M, C)
    hpt = jnp.swapaxes(higher_points, 1, 2)
    H = 1
    NS = N // H
    sc_fn = _make_sc_gather(NS, C, 8, 4)
    outs = []
    for b in range(B):
        for h in range(H):
            lp_s = lax.slice_in_dim(lower_points[b], h * NS, (h + 1) * NS)
            idx_s, w_s = _tc_stage_batch(lp_s, hpt[b], M, b * M)
            outs.append(sc_fn(feats_flat, idx_s.reshape(NS * 3),
                              w_s.reshape(NS, 48)))
    return jnp.concatenate(outs, axis=0).reshape(B, N, C)
